# Initial kernel scaffold; baseline (speedup 1.0000x reference)
import jax
import jax.numpy as jnp
from jax.experimental import pallas as pl


def kernel(x_politician, x_company, edge_index_p2c, edge_index_c2p, Wl1_pc, Wr1_pc, b1_pc, Wl1_cp, Wr1_cp, b1_cp, Wl2_pc, Wr2_pc, b2_pc, Wl2_cp, Wr2_cp, b2_cp):
    raise NotImplementedError("write your pallas kernel here")


# speedup vs baseline: 2.1612x; 2.1612x over previous
import functools

import jax
import jax.numpy as jnp
from jax import lax
from jax.experimental import pallas as pl
from jax.experimental.pallas import tpu as pltpu
from jax.experimental.pallas import tpu_sc as plsc

N = 10000
E = 160000
D_IN = 256
H = 512

NP = 10240
DUMMY = 10000
NS = 16
NC = 2
BLK = 128
EPAD = 163840
NBLK = EPAD // (NS * BLK)
HBLK = NBLK // 2
EROWS = EPAD // BLK
STRIPE = NP // NS


def _mesh():
  return plsc.VectorSubcoreMesh(
      core_axis_name="c", subcore_axis_name="s", num_cores=NC,
      num_subcores=NS)


@functools.lru_cache(maxsize=None)
def _make_seg_kernel(C):
  out_type = [
      jax.ShapeDtypeStruct((NP, C * 128), jnp.float32),
      jax.ShapeDtypeStruct((NP, C * 128), jnp.float32),
  ]
  scratch = {
      "acc": pltpu.VMEM_SHARED((NP, 128), jnp.float32),
      "sidx": pltpu.VMEM((HBLK, BLK), jnp.int32),
      "didx": pltpu.VMEM((HBLK, BLK), jnp.int32),
      "rows0": pltpu.VMEM((BLK, 128), jnp.float32),
      "rows1": pltpu.VMEM((BLK, 128), jnp.float32),
      "gsem0": pltpu.SemaphoreType.DMA,
      "gsem1": pltpu.SemaphoreType.DMA,
  }

  def body(xa, xb, sa, da, sb, db, z128, suma, sumb, sc):
    core = lax.axis_index("c")
    sid = lax.axis_index("s")
    stripe = pl.ds(sid * STRIPE, STRIPE)

    for x_flat, s_idx, d_idx, sum_out in (
        (xa, sa, da, suma), (xb, sb, db, sumb)):
      for k in range(C // 2):
        chunk = core + 2 * k
        pltpu.sync_copy(z128.at[stripe], sc["acc"].at[stripe])
        plsc.subcore_barrier()
        for q in range(2):
          base = sid * NBLK + q * HBLK
          pltpu.sync_copy(d_idx.at[pl.ds(base, HBLK)], sc["didx"])
          pltpu.sync_copy(
              s_idx.at[pl.ds(chunk * EROWS + base, HBLK)], sc["sidx"])

          pltpu.async_copy(
              x_flat.at[sc["sidx"].at[0]], sc["rows0"], sc["gsem0"])

          @pl.loop(0, HBLK, step=2)
          def _(j):
            for ph, (rows, sem, orows, osem) in enumerate((
                (sc["rows0"], sc["gsem0"], sc["rows1"], sc["gsem1"]),
                (sc["rows1"], sc["gsem1"], sc["rows0"], sc["gsem0"]))):
              jj = j + ph
              pltpu.make_async_copy(x_flat.at[sc["sidx"].at[jj]], rows,
                                    sem).wait()

              @pl.when(jj + 1 < HBLK)
              def _():
                pltpu.async_copy(
                    x_flat.at[sc["sidx"].at[jj + 1]], orows, osem)

              pltpu.sync_copy(rows, sc["acc"].at[sc["didx"].at[jj]],
                              add=True)

        plsc.subcore_barrier()
        pltpu.sync_copy(
            sc["acc"].at[stripe],
            sum_out.at[stripe, pl.ds(chunk * 128, 128)])
        plsc.subcore_barrier()

  def wrapped(*args):
    def body_with_scratch(*refs):
      body(*refs[:7], refs[7], refs[8], refs[9])

    return pl.kernel(
        body_with_scratch,
        out_type=out_type,
        mesh=_mesh(),
        scratch_types=[scratch],
    )(*args)

  return wrapped


@functools.lru_cache(maxsize=None)
def _make_cnt_kernel():
  out_type = [jax.ShapeDtypeStruct((NC * NP, 128), jnp.float32)]
  scratch = {
      "cnt": pltpu.VMEM_SHARED((NP, 128), jnp.float32),
      "didx": pltpu.VMEM((NBLK, BLK), jnp.int32),
      "ones_v": pltpu.VMEM((BLK, 128), jnp.float32),
  }

  def body(dab, o128, z128, cnt_out, sc):
    core = lax.axis_index("c")
    sid = lax.axis_index("s")
    stripe = pl.ds(sid * STRIPE, STRIPE)
    pltpu.sync_copy(o128, sc["ones_v"])
    pltpu.sync_copy(z128.at[stripe], sc["cnt"].at[stripe])
    pltpu.sync_copy(dab.at[pl.ds(core * EROWS + sid * NBLK, NBLK)],
                    sc["didx"])
    plsc.subcore_barrier()

    @pl.loop(0, NBLK)
    def _(j):
      pltpu.sync_copy(sc["ones_v"], sc["cnt"].at[sc["didx"].at[j]], add=True)

    plsc.subcore_barrier()
    pltpu.sync_copy(sc["cnt"].at[stripe],
                    cnt_out.at[pl.ds(core * NP + sid * STRIPE, STRIPE)])

  def wrapped(*args):
    def body_with_scratch(*refs):
      body(refs[0], refs[1], refs[2], refs[3], refs[4])

    return pl.kernel(
        body_with_scratch,
        out_type=out_type,
        mesh=_mesh(),
        scratch_types=[scratch],
    )(*args)

  return wrapped


def _combine_body(relu, sum_ref, cnt_ref, x_ref, wl_ref, wr_ref, b_ref, o_ref):
  recip = 1.0 / jnp.maximum(cnt_ref[...], 1.0)
  acc = jnp.dot(sum_ref[...] * recip, wl_ref[...],
                preferred_element_type=jnp.float32)
  acc = acc + jnp.dot(x_ref[...], wr_ref[...],
                      preferred_element_type=jnp.float32)
  acc = acc + b_ref[...]
  o_ref[...] = jnp.maximum(acc, 0.0) if relu else acc


def _combine(sums, cnt, x, wl, wr, b, relu):
  d = x.shape[1]
  rb = 512
  grid = (NP // rb,)
  return pl.pallas_call(
      functools.partial(_combine_body, relu),
      grid=grid,
      in_specs=[
          pl.BlockSpec((rb, d), lambda r: (r, 0)),
          pl.BlockSpec((rb, 1), lambda r: (r, 0)),
          pl.BlockSpec((rb, d), lambda r: (r, 0)),
          pl.BlockSpec((d, H), lambda r: (0, 0)),
          pl.BlockSpec((d, H), lambda r: (0, 0)),
          pl.BlockSpec((1, H), lambda r: (0, 0)),
      ],
      out_specs=pl.BlockSpec((rb, H), lambda r: (r, 0)),
      out_shape=jax.ShapeDtypeStruct((NP, H), jnp.float32),
      compiler_params=pltpu.CompilerParams(
          dimension_semantics=("parallel",)),
  )(sums, cnt, x, wl, wr, b)


def _pad_rows(x):
  return jnp.pad(x, ((0, NP - x.shape[0]), (0, 0)))


def _chunk_table(x, c):
  return x.reshape(NP, c, 128).transpose(1, 0, 2).reshape(c * NP, 128)


def _prep_idx(ei, chunks):
  src = ei[0].astype(jnp.int32)
  dst = ei[1].astype(jnp.int32)
  src_p = jnp.pad(src, (0, EPAD - E))
  dst_p = jnp.pad(dst, (0, EPAD - E), constant_values=DUMMY)
  outs = []
  for c in chunks:
    offs = jnp.arange(c, dtype=jnp.int32)[:, None] * NP
    outs.append((src_p[None, :] + offs).reshape(c * EROWS, BLK))
  outs.append(dst_p.reshape(EROWS, BLK))
  return outs


def kernel(x_politician, x_company, edge_index_p2c, edge_index_c2p,
           Wl1_pc, Wr1_pc, b1_pc, Wl1_cp, Wr1_cp, b1_cp,
           Wl2_pc, Wr2_pc, b2_pc, Wl2_cp, Wr2_cp, b2_cp):
  xp = _pad_rows(x_politician)
  xc = _pad_rows(x_company)
  sa1, sa2, da = _prep_idx(edge_index_p2c, (2, 4))
  sb1, sb2, db = _prep_idx(edge_index_c2p, (2, 4))

  z128 = jnp.zeros((NP, 128), jnp.float32)
  o128 = jnp.ones((BLK, 128), jnp.float32)

  (cntab,) = _make_cnt_kernel()(
      jnp.concatenate([da, db], axis=0), o128, z128)
  cnt_a = cntab[:NP, :1]
  cnt_b = cntab[NP:, :1]

  suma1, sumb1 = _make_seg_kernel(2)(
      _chunk_table(xp, 2), _chunk_table(xc, 2), sa1, da, sb1, db, z128)

  h_com = _combine(suma1, cnt_a, xc, Wl1_pc, Wr1_pc,
                   b1_pc.reshape(1, H), relu=True)
  h_pol = _combine(sumb1, cnt_b, xp, Wl1_cp, Wr1_cp,
                   b1_cp.reshape(1, H), relu=True)

  suma2, sumb2 = _make_seg_kernel(4)(
      _chunk_table(h_pol, 4), _chunk_table(h_com, 4), sa2, da, sb2, db, z128)

  z_com = _combine(suma2, cnt_a, h_com, Wl2_pc, Wr2_pc,
                   b2_pc.reshape(1, H), relu=False)
  z_pol = _combine(sumb2, cnt_b, h_pol, Wl2_cp, Wr2_cp,
                   b2_cp.reshape(1, H), relu=False)

  return (z_pol[:N], z_com[:N])


# speedup vs baseline: 2.3234x; 1.0750x over previous
import functools

import jax
import jax.numpy as jnp
from jax import lax
from jax.experimental import pallas as pl
from jax.experimental.pallas import tpu as pltpu
from jax.experimental.pallas import tpu_sc as plsc

N = 10000
E = 160000
D_IN = 256
H = 512

NP = 10240
DUMMY = 10000
NS = 16
NC = 2
BLK = 64
EPAD = 163840
NBLK = EPAD // (NS * BLK)
QBLK = NBLK // 4
EROWS = EPAD // BLK
STRIPE = NP // NS
NBUF = 4


def _mesh():
  return plsc.VectorSubcoreMesh(
      core_axis_name="c", subcore_axis_name="s", num_cores=NC,
      num_subcores=NS)


@functools.lru_cache(maxsize=None)
def _make_seg_kernel(C):
  out_type = [
      jax.ShapeDtypeStruct((NP, C * 128), jnp.float32),
      jax.ShapeDtypeStruct((NP, C * 128), jnp.float32),
  ]
  scratch = {
      "acc": pltpu.VMEM_SHARED((NP, 128), jnp.float32),
      "sidx": pltpu.VMEM((QBLK, BLK), jnp.int32),
      "didx": pltpu.VMEM((QBLK, BLK), jnp.int32),
  }
  for b in range(NBUF):
    scratch[f"rows{b}"] = pltpu.VMEM((BLK, 128), jnp.float32)
    scratch[f"gsem{b}"] = pltpu.SemaphoreType.DMA
    scratch[f"ssem{b}"] = pltpu.SemaphoreType.DMA

  def body(xa, xb, sa, da, sb, db, z128, suma, sumb, sc):
    core = lax.axis_index("c")
    sid = lax.axis_index("s")
    stripe = pl.ds(sid * STRIPE, STRIPE)
    rows = [sc[f"rows{b}"] for b in range(NBUF)]
    gsem = [sc[f"gsem{b}"] for b in range(NBUF)]
    ssem = [sc[f"ssem{b}"] for b in range(NBUF)]

    for x_flat, s_idx, d_idx, sum_out in (
        (xa, sa, da, suma), (xb, sb, db, sumb)):
      for k in range(C // 2):
        chunk = core + 2 * k
        pltpu.sync_copy(z128.at[stripe], sc["acc"].at[stripe])
        plsc.subcore_barrier()
        for q in range(4):
          base = sid * NBLK + q * QBLK
          pltpu.sync_copy(d_idx.at[pl.ds(base, QBLK)], sc["didx"])
          pltpu.sync_copy(
              s_idx.at[pl.ds(chunk * EROWS + base, QBLK)], sc["sidx"])

          pltpu.async_copy(x_flat.at[sc["sidx"].at[0]], rows[0], gsem[0])
          pltpu.async_copy(x_flat.at[sc["sidx"].at[1]], rows[1], gsem[1])

          @pl.loop(0, QBLK, step=NBUF)
          def _(j):
            for ph in range(NBUF):
              b = ph
              b2 = (ph + 2) % NBUF
              jj = j + ph
              pltpu.make_async_copy(x_flat.at[sc["sidx"].at[jj]], rows[b],
                                    gsem[b]).wait()
              pltpu.async_copy(rows[b], sc["acc"].at[sc["didx"].at[jj]],
                               ssem[b], add=True)

              @pl.when(jj >= 2)
              def _():
                pltpu.make_async_copy(
                    rows[b2], sc["acc"].at[sc["didx"].at[jj]],
                    ssem[b2]).wait()

              @pl.when(jj + 2 < QBLK)
              def _():
                pltpu.async_copy(x_flat.at[sc["sidx"].at[jj + 2]], rows[b2],
                                 gsem[b2])

          for jj in (QBLK - 2, QBLK - 1):
            b = jj % NBUF
            pltpu.make_async_copy(rows[b], sc["acc"].at[sc["didx"].at[jj]],
                                  ssem[b]).wait()

        plsc.subcore_barrier()
        pltpu.sync_copy(
            sc["acc"].at[stripe],
            sum_out.at[stripe, pl.ds(chunk * 128, 128)])
        plsc.subcore_barrier()

  def wrapped(*args):
    def body_with_scratch(*refs):
      body(*refs[:7], refs[7], refs[8], refs[9])

    return pl.kernel(
        body_with_scratch,
        out_type=out_type,
        mesh=_mesh(),
        scratch_types=[scratch],
    )(*args)

  return wrapped


@functools.lru_cache(maxsize=None)
def _make_cnt_kernel():
  out_type = [jax.ShapeDtypeStruct((NC * NP, 128), jnp.float32)]
  scratch = {
      "cnt": pltpu.VMEM_SHARED((NP, 128), jnp.float32),
      "didx": pltpu.VMEM((NBLK, BLK), jnp.int32),
      "ones_v": pltpu.VMEM((BLK, 128), jnp.float32),
  }

  def body(dab, o128, z128, cnt_out, sc):
    core = lax.axis_index("c")
    sid = lax.axis_index("s")
    stripe = pl.ds(sid * STRIPE, STRIPE)
    pltpu.sync_copy(o128, sc["ones_v"])
    pltpu.sync_copy(z128.at[stripe], sc["cnt"].at[stripe])
    pltpu.sync_copy(dab.at[pl.ds(core * EROWS + sid * NBLK, NBLK)],
                    sc["didx"])
    plsc.subcore_barrier()

    @pl.loop(0, NBLK)
    def _(j):
      pltpu.sync_copy(sc["ones_v"], sc["cnt"].at[sc["didx"].at[j]], add=True)

    plsc.subcore_barrier()
    pltpu.sync_copy(sc["cnt"].at[stripe],
                    cnt_out.at[pl.ds(core * NP + sid * STRIPE, STRIPE)])

  def wrapped(*args):
    def body_with_scratch(*refs):
      body(refs[0], refs[1], refs[2], refs[3], refs[4])

    return pl.kernel(
        body_with_scratch,
        out_type=out_type,
        mesh=_mesh(),
        scratch_types=[scratch],
    )(*args)

  return wrapped


def _combine_body(relu, sum_ref, cnt_ref, x_ref, wl_ref, wr_ref, b_ref, o_ref):
  recip = 1.0 / jnp.maximum(cnt_ref[...], 1.0)
  acc = jnp.dot(sum_ref[...] * recip, wl_ref[...],
                preferred_element_type=jnp.float32)
  acc = acc + jnp.dot(x_ref[...], wr_ref[...],
                      preferred_element_type=jnp.float32)
  acc = acc + b_ref[...]
  o_ref[...] = jnp.maximum(acc, 0.0) if relu else acc


def _combine(sums, cnt, x, wl, wr, b, relu):
  d = x.shape[1]
  rb = 512
  grid = (NP // rb,)
  return pl.pallas_call(
      functools.partial(_combine_body, relu),
      grid=grid,
      in_specs=[
          pl.BlockSpec((rb, d), lambda r: (r, 0)),
          pl.BlockSpec((rb, 1), lambda r: (r, 0)),
          pl.BlockSpec((rb, d), lambda r: (r, 0)),
          pl.BlockSpec((d, H), lambda r: (0, 0)),
          pl.BlockSpec((d, H), lambda r: (0, 0)),
          pl.BlockSpec((1, H), lambda r: (0, 0)),
      ],
      out_specs=pl.BlockSpec((rb, H), lambda r: (r, 0)),
      out_shape=jax.ShapeDtypeStruct((NP, H), jnp.float32),
      compiler_params=pltpu.CompilerParams(
          dimension_semantics=("parallel",)),
  )(sums, cnt, x, wl, wr, b)


def _pad_rows(x):
  return jnp.pad(x, ((0, NP - x.shape[0]), (0, 0)))


def _chunk_table(x, c):
  return x.reshape(NP, c, 128).transpose(1, 0, 2).reshape(c * NP, 128)


def _prep_idx(ei, chunks):
  src = ei[0].astype(jnp.int32)
  dst = ei[1].astype(jnp.int32)
  src_p = jnp.pad(src, (0, EPAD - E))
  dst_p = jnp.pad(dst, (0, EPAD - E), constant_values=DUMMY)
  outs = []
  for c in chunks:
    offs = jnp.arange(c, dtype=jnp.int32)[:, None] * NP
    outs.append((src_p[None, :] + offs).reshape(c * EROWS, BLK))
  outs.append(dst_p.reshape(EROWS, BLK))
  return outs


def kernel(x_politician, x_company, edge_index_p2c, edge_index_c2p,
           Wl1_pc, Wr1_pc, b1_pc, Wl1_cp, Wr1_cp, b1_cp,
           Wl2_pc, Wr2_pc, b2_pc, Wl2_cp, Wr2_cp, b2_cp):
  xp = _pad_rows(x_politician)
  xc = _pad_rows(x_company)
  sa1, sa2, da = _prep_idx(edge_index_p2c, (2, 4))
  sb1, sb2, db = _prep_idx(edge_index_c2p, (2, 4))

  z128 = jnp.zeros((NP, 128), jnp.float32)
  o128 = jnp.ones((BLK, 128), jnp.float32)

  (cntab,) = _make_cnt_kernel()(
      jnp.concatenate([da, db], axis=0), o128, z128)
  cnt_a = cntab[:NP, :1]
  cnt_b = cntab[NP:, :1]

  suma1, sumb1 = _make_seg_kernel(2)(
      _chunk_table(xp, 2), _chunk_table(xc, 2), sa1, da, sb1, db, z128)

  h_com = _combine(suma1, cnt_a, xc, Wl1_pc, Wr1_pc,
                   b1_pc.reshape(1, H), relu=True)
  h_pol = _combine(sumb1, cnt_b, xp, Wl1_cp, Wr1_cp,
                   b1_cp.reshape(1, H), relu=True)

  suma2, sumb2 = _make_seg_kernel(4)(
      _chunk_table(h_pol, 4), _chunk_table(h_com, 4), sa2, da, sb2, db, z128)

  z_com = _combine(suma2, cnt_a, h_com, Wl2_pc, Wr2_pc,
                   b2_pc.reshape(1, H), relu=False)
  z_pol = _combine(sumb2, cnt_b, h_pol, Wl2_cp, Wr2_cp,
                   b2_cp.reshape(1, H), relu=False)

  return (z_pol[:N], z_com[:N])


# speedup vs baseline: 2.3891x; 1.0283x over previous
import functools

import jax
import jax.numpy as jnp
from jax import lax
from jax.experimental import pallas as pl
from jax.experimental.pallas import tpu as pltpu
from jax.experimental.pallas import tpu_sc as plsc

N = 10000
E = 160000
D_IN = 256
H = 512

NP = 10240
DUMMY = 10000
NS = 16
NC = 2
BLK = 64
EPAD = 163840
NBLK = EPAD // (NS * BLK)
QBLK = NBLK // 4
EROWS = EPAD // BLK
STRIPE = NP // NS
NBUF = 4


def _mesh():
  return plsc.VectorSubcoreMesh(
      core_axis_name="c", subcore_axis_name="s", num_cores=NC,
      num_subcores=NS)


@functools.lru_cache(maxsize=None)
def _make_seg_kernel(C):
  out_type = [
      jax.ShapeDtypeStruct((NP, C * 128), jnp.float32),
      jax.ShapeDtypeStruct((NP, C * 128), jnp.float32),
  ]
  scratch = {
      "acc": pltpu.VMEM_SHARED((NP, 128), jnp.float32),
      "sidx": pltpu.VMEM((QBLK, BLK), jnp.int32),
      "didx": pltpu.VMEM((QBLK, BLK), jnp.int32),
  }
  for b in range(NBUF):
    scratch[f"rows{b}"] = pltpu.VMEM((BLK, 128), jnp.float32)
    scratch[f"gsem{b}"] = pltpu.SemaphoreType.DMA

  def body(xa, xb, sa, da, sb, db, z128, suma, sumb, sc):
    core = lax.axis_index("c")
    sid = lax.axis_index("s")
    stripe = pl.ds(sid * STRIPE, STRIPE)
    rows = [sc[f"rows{b}"] for b in range(NBUF)]
    gsem = [sc[f"gsem{b}"] for b in range(NBUF)]

    for x_flat, s_idx, d_idx, sum_out in (
        (xa, sa, da, suma), (xb, sb, db, sumb)):
      for k in range(C // 2):
        chunk = core + 2 * k
        pltpu.sync_copy(z128.at[stripe], sc["acc"].at[stripe])
        plsc.subcore_barrier()
        for q in range(4):
          base = sid * NBLK + q * QBLK
          pltpu.sync_copy(d_idx.at[pl.ds(base, QBLK)], sc["didx"])
          pltpu.sync_copy(
              s_idx.at[pl.ds(chunk * EROWS + base, QBLK)], sc["sidx"])

          for p in range(NBUF - 1):
            pltpu.async_copy(x_flat.at[sc["sidx"].at[p]], rows[p], gsem[p])

          @pl.loop(0, QBLK, step=NBUF)
          def _(j):
            for ph in range(NBUF):
              b = ph
              b2 = (ph + NBUF - 1) % NBUF
              jj = j + ph
              pltpu.make_async_copy(x_flat.at[sc["sidx"].at[jj]], rows[b],
                                    gsem[b]).wait()
              pltpu.sync_copy(rows[b], sc["acc"].at[sc["didx"].at[jj]],
                              add=True)

              @pl.when(jj + NBUF - 1 < QBLK)
              def _():
                pltpu.async_copy(
                    x_flat.at[sc["sidx"].at[jj + NBUF - 1]], rows[b2],
                    gsem[b2])

        plsc.subcore_barrier()
        pltpu.sync_copy(
            sc["acc"].at[stripe],
            sum_out.at[stripe, pl.ds(chunk * 128, 128)])
        plsc.subcore_barrier()

  def wrapped(*args):
    def body_with_scratch(*refs):
      body(*refs[:7], refs[7], refs[8], refs[9])

    return pl.kernel(
        body_with_scratch,
        out_type=out_type,
        mesh=_mesh(),
        scratch_types=[scratch],
    )(*args)

  return wrapped


@functools.lru_cache(maxsize=None)
def _make_cnt_kernel():
  out_type = [jax.ShapeDtypeStruct((NC * NP, 128), jnp.float32)]
  scratch = {
      "cnt": pltpu.VMEM_SHARED((NP, 128), jnp.float32),
      "didx": pltpu.VMEM((NBLK, BLK), jnp.int32),
      "ones_v": pltpu.VMEM((BLK, 128), jnp.float32),
  }

  def body(dab, o128, z128, cnt_out, sc):
    core = lax.axis_index("c")
    sid = lax.axis_index("s")
    stripe = pl.ds(sid * STRIPE, STRIPE)
    pltpu.sync_copy(o128, sc["ones_v"])
    pltpu.sync_copy(z128.at[stripe], sc["cnt"].at[stripe])
    pltpu.sync_copy(dab.at[pl.ds(core * EROWS + sid * NBLK, NBLK)],
                    sc["didx"])
    plsc.subcore_barrier()

    @pl.loop(0, NBLK)
    def _(j):
      pltpu.sync_copy(sc["ones_v"], sc["cnt"].at[sc["didx"].at[j]], add=True)

    plsc.subcore_barrier()
    pltpu.sync_copy(sc["cnt"].at[stripe],
                    cnt_out.at[pl.ds(core * NP + sid * STRIPE, STRIPE)])

  def wrapped(*args):
    def body_with_scratch(*refs):
      body(refs[0], refs[1], refs[2], refs[3], refs[4])

    return pl.kernel(
        body_with_scratch,
        out_type=out_type,
        mesh=_mesh(),
        scratch_types=[scratch],
    )(*args)

  return wrapped


def _combine_body(relu, sum_ref, cnt_ref, x_ref, wl_ref, wr_ref, b_ref, o_ref):
  recip = 1.0 / jnp.maximum(cnt_ref[...], 1.0)
  acc = jnp.dot(sum_ref[...] * recip, wl_ref[...],
                preferred_element_type=jnp.float32)
  acc = acc + jnp.dot(x_ref[...], wr_ref[...],
                      preferred_element_type=jnp.float32)
  acc = acc + b_ref[...]
  o_ref[...] = jnp.maximum(acc, 0.0) if relu else acc


def _combine(sums, cnt, x, wl, wr, b, relu):
  d = x.shape[1]
  rb = 512
  grid = (NP // rb,)
  return pl.pallas_call(
      functools.partial(_combine_body, relu),
      grid=grid,
      in_specs=[
          pl.BlockSpec((rb, d), lambda r: (r, 0)),
          pl.BlockSpec((rb, 1), lambda r: (r, 0)),
          pl.BlockSpec((rb, d), lambda r: (r, 0)),
          pl.BlockSpec((d, H), lambda r: (0, 0)),
          pl.BlockSpec((d, H), lambda r: (0, 0)),
          pl.BlockSpec((1, H), lambda r: (0, 0)),
      ],
      out_specs=pl.BlockSpec((rb, H), lambda r: (r, 0)),
      out_shape=jax.ShapeDtypeStruct((NP, H), jnp.float32),
      compiler_params=pltpu.CompilerParams(
          dimension_semantics=("parallel",)),
  )(sums, cnt, x, wl, wr, b)


def _pad_rows(x):
  return jnp.pad(x, ((0, NP - x.shape[0]), (0, 0)))


def _chunk_table(x, c):
  return x.reshape(NP, c, 128).transpose(1, 0, 2).reshape(c * NP, 128)


def _prep_idx(ei, chunks):
  src = ei[0].astype(jnp.int32)
  dst = ei[1].astype(jnp.int32)
  src_p = jnp.pad(src, (0, EPAD - E))
  dst_p = jnp.pad(dst, (0, EPAD - E), constant_values=DUMMY)
  outs = []
  for c in chunks:
    offs = jnp.arange(c, dtype=jnp.int32)[:, None] * NP
    outs.append((src_p[None, :] + offs).reshape(c * EROWS, BLK))
  outs.append(dst_p.reshape(EROWS, BLK))
  return outs


def kernel(x_politician, x_company, edge_index_p2c, edge_index_c2p,
           Wl1_pc, Wr1_pc, b1_pc, Wl1_cp, Wr1_cp, b1_cp,
           Wl2_pc, Wr2_pc, b2_pc, Wl2_cp, Wr2_cp, b2_cp):
  xp = _pad_rows(x_politician)
  xc = _pad_rows(x_company)
  sa1, sa2, da = _prep_idx(edge_index_p2c, (2, 4))
  sb1, sb2, db = _prep_idx(edge_index_c2p, (2, 4))

  z128 = jnp.zeros((NP, 128), jnp.float32)
  o128 = jnp.ones((BLK, 128), jnp.float32)

  (cntab,) = _make_cnt_kernel()(
      jnp.concatenate([da, db], axis=0), o128, z128)
  cnt_a = cntab[:NP, :1]
  cnt_b = cntab[NP:, :1]

  suma1, sumb1 = _make_seg_kernel(2)(
      _chunk_table(xp, 2), _chunk_table(xc, 2), sa1, da, sb1, db, z128)

  h_com = _combine(suma1, cnt_a, xc, Wl1_pc, Wr1_pc,
                   b1_pc.reshape(1, H), relu=True)
  h_pol = _combine(sumb1, cnt_b, xp, Wl1_cp, Wr1_cp,
                   b1_cp.reshape(1, H), relu=True)

  suma2, sumb2 = _make_seg_kernel(4)(
      _chunk_table(h_pol, 4), _chunk_table(h_com, 4), sa2, da, sb2, db, z128)

  z_com = _combine(suma2, cnt_a, h_com, Wl2_pc, Wr2_pc,
                   b2_pc.reshape(1, H), relu=False)
  z_pol = _combine(sumb2, cnt_b, h_pol, Wl2_cp, Wr2_cp,
                   b2_cp.reshape(1, H), relu=False)

  return (z_pol[:N], z_com[:N])
